# baseline (speedup 1.0000x reference)
import jax
import jax.numpy as jnp
from jax import lax
from jax.experimental import pallas as pl
from jax.experimental.pallas import tpu as pltpu
from jax.experimental.pallas import tpu_sc as plsc

BATCH = 16384
EMBED_DIM = 32
NC = 2
NS = 16
NW = NC * NS
BPW = BATCH // NW
GCH = 128


def _sc_body(uidx_hbm, pidx_hbm, uemb_hbm, ub_hbm, pemb_hbm, pb_hbm,
             partials_hbm, bsum_hbm,
             uidx_v, pidx_v, usup_v, psup_v, urows_v, prows_v,
             ubv, pbv, bsumv, accv, sems):
    wid = lax.axis_index("c") * NS + lax.axis_index("s")
    base = wid * BPW
    pltpu.sync_copy(uidx_hbm.at[pl.ds(base, BPW)], uidx_v)
    pltpu.sync_copy(pidx_hbm.at[pl.ds(base, BPW)], pidx_v)

    bias_cps = []
    for j in range(BPW // GCH):
        s = pl.ds(j * GCH, GCH)
        bias_cps.append(pltpu.async_copy(ub_hbm.at[uidx_v.at[s]], ubv.at[s], sems.at[2]))
        bias_cps.append(pltpu.async_copy(pb_hbm.at[pidx_v.at[s]], pbv.at[s], sems.at[2]))

    def sup_body(i, carry):
        s = pl.ds(i * 16, 16)
        usup_v[s] = lax.shift_right_logical(uidx_v[s], 2)
        psup_v[s] = lax.shift_right_logical(pidx_v[s], 2)
        return carry
    lax.fori_loop(0, BPW // 16, sup_body, 0)

    def fire(c, buf):
        s = pl.ds(c * GCH, GCH)
        pltpu.async_copy(uemb_hbm.at[usup_v.at[s]], urows_v.at[buf], sems.at[buf])
        pltpu.async_copy(pemb_hbm.at[psup_v.at[s]], prows_v.at[buf], sems.at[buf])

    fire(0, 0)

    def chunk_body(c, acc):
        buf = lax.rem(c, 2)

        @pl.when(c + 1 < BPW // GCH)
        def _():
            fire(c + 1, lax.rem(c + 1, 2))

        pltpu.make_async_copy(uemb_hbm.at[pl.ds(0, GCH)], urows_v.at[buf], sems.at[buf]).wait()
        pltpu.make_async_copy(pemb_hbm.at[pl.ds(0, GCH)], prows_v.at[buf], sems.at[buf]).wait()

        def dot_body(i, a):
            uvec = uidx_v[pl.ds(c * GCH + i * 8, 8)]
            pvec = pidx_v[pl.ds(c * GCH + i * 8, 8)]
            for t in range(8):
                cu = (uvec[t] & 3) * EMBED_DIM
                cp_ = (pvec[t] & 3) * EMBED_DIM
                k = i * 8 + t
                a = a + urows_v[buf, k, pl.ds(cu, 16)] * prows_v[buf, k, pl.ds(cp_, 16)]
                a = a + urows_v[buf, k, pl.ds(cu + 16, 16)] * prows_v[buf, k, pl.ds(cp_ + 16, 16)]
            return a
        return lax.fori_loop(0, GCH // 8, dot_body, acc)

    acc = lax.fori_loop(0, BPW // GCH, chunk_body, jnp.zeros((16,), jnp.float32))
    accv[...] = acc

    for cb in bias_cps:
        cb.wait()

    def bias_body(i, carry):
        s = pl.ds(i * 16, 16)
        bsumv[s] = ubv[s] + pbv[s]
        return carry
    lax.fori_loop(0, BPW // 16, bias_body, 0)

    pltpu.sync_copy(accv, partials_hbm.at[wid])
    pltpu.sync_copy(bsumv, bsum_hbm.at[pl.ds(base, BPW)])


def _sc_stage(u_idx, p_idx, uemb2, ub_flat, pemb2, pb_flat):
    mesh = plsc.VectorSubcoreMesh(core_axis_name="c", subcore_axis_name="s")
    f = pl.kernel(
        _sc_body,
        mesh=mesh,
        out_type=[
            jax.ShapeDtypeStruct((NW, 16), jnp.float32),
            jax.ShapeDtypeStruct((BATCH,), jnp.float32),
        ],
        scratch_types=[
            pltpu.VMEM((BPW,), jnp.int32),
            pltpu.VMEM((BPW,), jnp.int32),
            pltpu.VMEM((BPW,), jnp.int32),
            pltpu.VMEM((BPW,), jnp.int32),
            pltpu.VMEM((2, GCH, 128), jnp.float32),
            pltpu.VMEM((2, GCH, 128), jnp.float32),
            pltpu.VMEM((BPW,), jnp.float32),
            pltpu.VMEM((BPW,), jnp.float32),
            pltpu.VMEM((BPW,), jnp.float32),
            pltpu.VMEM((16,), jnp.float32),
            pltpu.SemaphoreType.DMA((3,)),
        ],
    )
    return f(u_idx, p_idx, uemb2, ub_flat, pemb2, pb_flat)


def _tc_final(partials_ref, bsum_ref, out_ref):
    s = jnp.sum(partials_ref[...])
    out_ref[...] = jax.nn.sigmoid(bsum_ref[...] + s)


def kernel(inputs, user_embedding, user_bias, place_embedding, place_bias):
    u_idx = inputs[:, 0]
    p_idx = inputs[:, 1]
    partials, bsum = _sc_stage(
        u_idx, p_idx, user_embedding.reshape(-1, 128), user_bias[:, 0],
        place_embedding.reshape(-1, 128), place_bias[:, 0])
    out = pl.pallas_call(
        _tc_final,
        out_shape=jax.ShapeDtypeStruct((128, 128), jnp.float32),
    )(partials, bsum.reshape(128, 128))
    return out.reshape(BATCH, 1)


# speedup vs baseline: 1.0095x; 1.0095x over previous
import jax
import jax.numpy as jnp
from jax import lax
from jax.experimental import pallas as pl
from jax.experimental.pallas import tpu as pltpu
from jax.experimental.pallas import tpu_sc as plsc

BATCH = 16384
EMBED_DIM = 32
NC = 2
NS = 16
NW = NC * NS
BPW = BATCH // NW
GCH = 128


def _sc_body(uidx_hbm, pidx_hbm, uemb_hbm, ub_hbm, pemb_hbm, pb_hbm,
             partials_hbm, bsum_hbm,
             uidx_v, pidx_v, urows_v, prows_v, ubv, pbv, bsumv, accv, sems):
    wid = lax.axis_index("c") * NS + lax.axis_index("s")
    base = wid * BPW
    pltpu.sync_copy(uidx_hbm.at[pl.ds(base, BPW)], uidx_v)
    pltpu.sync_copy(pidx_hbm.at[pl.ds(base, BPW)], pidx_v)

    bias_cps = []
    for j in range(BPW // GCH):
        s = pl.ds(j * GCH, GCH)
        bias_cps.append(pltpu.async_copy(ub_hbm.at[uidx_v.at[s]], ubv.at[s], sems.at[2]))
        bias_cps.append(pltpu.async_copy(pb_hbm.at[pidx_v.at[s]], pbv.at[s], sems.at[2]))

    def fire(c, buf):
        s = pl.ds(c * GCH, GCH)
        pltpu.async_copy(uemb_hbm.at[uidx_v.at[s]], urows_v.at[buf], sems.at[buf])
        pltpu.async_copy(pemb_hbm.at[pidx_v.at[s]], prows_v.at[buf], sems.at[buf])

    fire(0, 0)

    def chunk_body(c, acc):
        buf = lax.rem(c, 2)

        @pl.when(c + 1 < BPW // GCH)
        def _():
            fire(c + 1, lax.rem(c + 1, 2))

        pltpu.make_async_copy(uemb_hbm.at[pl.ds(0, GCH)], urows_v.at[buf], sems.at[buf]).wait()
        pltpu.make_async_copy(pemb_hbm.at[pl.ds(0, GCH)], prows_v.at[buf], sems.at[buf]).wait()

        def dot_body(i, a):
            r = i * 4
            for t in range(4):
                a = a + urows_v[buf, r + t, pl.ds(0, 16)] * prows_v[buf, r + t, pl.ds(0, 16)]
                a = a + urows_v[buf, r + t, pl.ds(16, 16)] * prows_v[buf, r + t, pl.ds(16, 16)]
            return a
        return lax.fori_loop(0, GCH // 4, dot_body, acc)

    acc = lax.fori_loop(0, BPW // GCH, chunk_body, jnp.zeros((16,), jnp.float32))
    accv[...] = acc

    for cb in bias_cps:
        cb.wait()

    def bias_body(i, carry):
        s = pl.ds(i * 16, 16)
        bsumv[s] = ubv[s] + pbv[s]
        return carry
    lax.fori_loop(0, BPW // 16, bias_body, 0)

    pltpu.sync_copy(accv, partials_hbm.at[wid])
    pltpu.sync_copy(bsumv, bsum_hbm.at[pl.ds(base, BPW)])


def _sc_stage(u_idx, p_idx, uemb, ub_flat, pemb, pb_flat):
    mesh = plsc.VectorSubcoreMesh(core_axis_name="c", subcore_axis_name="s")
    f = pl.kernel(
        _sc_body,
        mesh=mesh,
        compiler_params=pltpu.CompilerParams(use_tc_tiling_on_sc=False),
        out_type=[
            jax.ShapeDtypeStruct((NW, 16), jnp.float32),
            jax.ShapeDtypeStruct((BATCH,), jnp.float32),
        ],
        scratch_types=[
            pltpu.VMEM((BPW,), jnp.int32),
            pltpu.VMEM((BPW,), jnp.int32),
            pltpu.VMEM((2, GCH, EMBED_DIM), jnp.float32),
            pltpu.VMEM((2, GCH, EMBED_DIM), jnp.float32),
            pltpu.VMEM((BPW,), jnp.float32),
            pltpu.VMEM((BPW,), jnp.float32),
            pltpu.VMEM((BPW,), jnp.float32),
            pltpu.VMEM((16,), jnp.float32),
            pltpu.SemaphoreType.DMA((3,)),
        ],
    )
    return f(u_idx, p_idx, uemb, ub_flat, pemb, pb_flat)


def _tc_final(partials_ref, bsum_ref, out_ref):
    s = jnp.sum(partials_ref[...])
    out_ref[...] = jax.nn.sigmoid(bsum_ref[...] + s)


def kernel(inputs, user_embedding, user_bias, place_embedding, place_bias):
    u_idx = inputs[:, 0]
    p_idx = inputs[:, 1]
    partials, bsum = _sc_stage(
        u_idx, p_idx, user_embedding, user_bias[:, 0],
        place_embedding, place_bias[:, 0])
    out = pl.pallas_call(
        _tc_final,
        out_shape=jax.ShapeDtypeStruct((128, 128), jnp.float32),
    )(partials, bsum.reshape(128, 128))
    return out.reshape(BATCH, 1)
